# baseline (speedup 1.0000x reference)
import functools

import jax
import jax.numpy as jnp
from jax import lax
from jax.experimental import pallas as pl
from jax.experimental.pallas import tpu as pltpu
from jax.experimental.pallas import tpu_sc as plsc

_D = 32
_IDXW = 128
_CHUNK = 1024


def _make_gather(B):
    info = plsc.get_sparse_core_info()
    nc, ns = info.num_cores, info.num_subcores
    nw = nc * ns
    b_per_w = B // nw
    k = _CHUNK // _IDXW
    n_chunks = b_per_w // _CHUNK
    rows_per_w = b_per_w // _IDXW
    assert n_chunks >= 4 and n_chunks % 2 == 0
    mesh = plsc.VectorSubcoreMesh(core_axis_name="c", subcore_axis_name="s")

    @functools.partial(
        pl.kernel,
        mesh=mesh,
        out_type=jax.ShapeDtypeStruct((B, _D), jnp.float32),
        compiler_params=pltpu.CompilerParams(use_tc_tiling_on_sc=False),
        scratch_types=[
            pltpu.VMEM((_CHUNK,), jnp.int32),
            pltpu.VMEM((_CHUNK,), jnp.int32),
            pltpu.VMEM((_CHUNK, _D), jnp.float32),
            pltpu.VMEM((_CHUNK, _D), jnp.float32),
            pltpu.SemaphoreType.DMA,
            pltpu.SemaphoreType.DMA,
            pltpu.SemaphoreType.DMA,
            pltpu.SemaphoreType.DMA,
            pltpu.SemaphoreType.DMA,
            pltpu.SemaphoreType.DMA,
        ],
    )
    def gather_kernel(idx_hbm, table_hbm, out_hbm,
                      idx_v0, idx_v1, rows_v0, rows_v1,
                      isem0, isem1, gsem0, gsem1, wsem0, wsem1):
        wid = lax.axis_index("s") * nc + lax.axis_index("c")
        row_base = wid * rows_per_w
        out_base = wid * b_per_w
        idx_bufs = (idx_v0, idx_v1)
        rows_bufs = (rows_v0, rows_v1)
        isems = (isem0, isem1)
        gsems = (gsem0, gsem1)
        wsems = (wsem0, wsem1)

        def load_idx(i, s):
            pltpu.async_copy(idx_hbm.at[pl.ds(out_base + i * _CHUNK, _CHUNK)],
                             idx_bufs[s], isems[s])

        def wait_idx(s):
            pltpu.make_async_copy(idx_hbm.at[pl.ds(out_base, _CHUNK)],
                                  idx_bufs[s], isems[s]).wait()

        def fire_gathers(s):
            pltpu.async_copy(
                table_hbm.at[idx_bufs[s]],
                rows_bufs[s],
                gsems[s],
            )

        def drain_gathers(s):
            pltpu.make_async_copy(
                table_hbm.at[idx_bufs[s]],
                rows_bufs[s],
                gsems[s],
            ).wait()

        def start_wb(i, s):
            pltpu.async_copy(
                rows_bufs[s],
                out_hbm.at[pl.ds(out_base + i * _CHUNK, _CHUNK)],
                wsems[s],
            )

        def wait_wb(s):
            pltpu.make_async_copy(
                rows_bufs[s], out_hbm.at[pl.ds(out_base, _CHUNK)],
                wsems[s]).wait()

        def step(i, s, first):
            p = 1 - s
            wait_idx(s)
            if not first:
                wait_wb(s)
            fire_gathers(s)
            drain_gathers(p)
            load_idx(i + 1, p)
            start_wb(i - 1, p)

        load_idx(0, 0)
        load_idx(1, 1)
        wait_idx(0)
        fire_gathers(0)
        step(1, 1, True)
        step(2, 0, False)

        def body(t, carry):
            step(3 + 2 * t, 1, False)
            step(4 + 2 * t, 0, False)
            return carry

        lax.fori_loop(0, (n_chunks - 4) // 2, body, 0)

        s = 1
        p = 0
        wait_idx(s)
        wait_wb(s)
        fire_gathers(s)
        drain_gathers(p)
        start_wb(n_chunks - 2, p)
        drain_gathers(s)
        start_wb(n_chunks - 1, s)
        wait_wb(p)
        wait_wb(s)

    return gather_kernel


def kernel(token_ids, weight):
    shape = token_ids.shape
    b = token_ids.size
    idx1d = token_ids.reshape(b).astype(jnp.int32)
    out = _make_gather(b)(idx1d, weight)
    return out.reshape(*shape, _D)


# speedup vs baseline: 1.0008x; 1.0008x over previous
import functools

import jax
import jax.numpy as jnp
from jax import lax
from jax.experimental import pallas as pl
from jax.experimental.pallas import tpu as pltpu
from jax.experimental.pallas import tpu_sc as plsc

_D = 32
_IDXW = 128
_CHUNK = 1600


def _make_gather(B):
    info = plsc.get_sparse_core_info()
    nc, ns = info.num_cores, info.num_subcores
    nw = nc * ns
    b_per_w = B // nw
    k = _CHUNK // _IDXW
    n_chunks = b_per_w // _CHUNK
    rows_per_w = b_per_w // _IDXW
    assert n_chunks >= 4 and n_chunks % 2 == 0
    mesh = plsc.VectorSubcoreMesh(core_axis_name="c", subcore_axis_name="s")

    @functools.partial(
        pl.kernel,
        mesh=mesh,
        out_type=jax.ShapeDtypeStruct((B, _D), jnp.float32),
        compiler_params=pltpu.CompilerParams(use_tc_tiling_on_sc=False),
        scratch_types=[
            pltpu.VMEM((_CHUNK,), jnp.int32),
            pltpu.VMEM((_CHUNK,), jnp.int32),
            pltpu.VMEM((_CHUNK, _D), jnp.float32),
            pltpu.VMEM((_CHUNK, _D), jnp.float32),
            pltpu.SemaphoreType.DMA,
            pltpu.SemaphoreType.DMA,
            pltpu.SemaphoreType.DMA,
            pltpu.SemaphoreType.DMA,
            pltpu.SemaphoreType.DMA,
            pltpu.SemaphoreType.DMA,
        ],
    )
    def gather_kernel(idx_hbm, table_hbm, out_hbm,
                      idx_v0, idx_v1, rows_v0, rows_v1,
                      isem0, isem1, gsem0, gsem1, wsem0, wsem1):
        wid = lax.axis_index("s") * nc + lax.axis_index("c")
        row_base = wid * rows_per_w
        out_base = wid * b_per_w
        idx_bufs = (idx_v0, idx_v1)
        rows_bufs = (rows_v0, rows_v1)
        isems = (isem0, isem1)
        gsems = (gsem0, gsem1)
        wsems = (wsem0, wsem1)

        def load_idx(i, s):
            pltpu.async_copy(idx_hbm.at[pl.ds(out_base + i * _CHUNK, _CHUNK)],
                             idx_bufs[s], isems[s])

        def wait_idx(s):
            pltpu.make_async_copy(idx_hbm.at[pl.ds(out_base, _CHUNK)],
                                  idx_bufs[s], isems[s]).wait()

        def fire_gathers(s):
            pltpu.async_copy(
                table_hbm.at[idx_bufs[s]],
                rows_bufs[s],
                gsems[s],
            )

        def drain_gathers(s):
            pltpu.make_async_copy(
                table_hbm.at[idx_bufs[s]],
                rows_bufs[s],
                gsems[s],
            ).wait()

        def start_wb(i, s):
            pltpu.async_copy(
                rows_bufs[s],
                out_hbm.at[pl.ds(out_base + i * _CHUNK, _CHUNK)],
                wsems[s],
            )

        def wait_wb(s):
            pltpu.make_async_copy(
                rows_bufs[s], out_hbm.at[pl.ds(out_base, _CHUNK)],
                wsems[s]).wait()

        def step(i, s, first):
            p = 1 - s
            wait_idx(s)
            if not first:
                wait_wb(s)
            fire_gathers(s)
            drain_gathers(p)
            load_idx(i + 1, p)
            start_wb(i - 1, p)

        load_idx(0, 0)
        load_idx(1, 1)
        wait_idx(0)
        fire_gathers(0)
        step(1, 1, True)
        step(2, 0, False)

        def body(t, carry):
            step(3 + 2 * t, 1, False)
            step(4 + 2 * t, 0, False)
            return carry

        lax.fori_loop(0, (n_chunks - 4) // 2, body, 0)

        s = 1
        p = 0
        wait_idx(s)
        wait_wb(s)
        fire_gathers(s)
        drain_gathers(p)
        start_wb(n_chunks - 2, p)
        drain_gathers(s)
        start_wb(n_chunks - 1, s)
        wait_wb(p)
        wait_wb(s)

    return gather_kernel


def kernel(token_ids, weight):
    shape = token_ids.shape
    b = token_ids.size
    idx1d = token_ids.reshape(b).astype(jnp.int32)
    out = _make_gather(b)(idx1d, weight)
    return out.reshape(*shape, _D)


# speedup vs baseline: 1.0008x; 1.0001x over previous
import functools

import jax
import jax.numpy as jnp
from jax import lax
from jax.experimental import pallas as pl
from jax.experimental.pallas import tpu as pltpu
from jax.experimental.pallas import tpu_sc as plsc

_D = 32
_CHUNK = 1600


def _make_gather(B):
    info = plsc.get_sparse_core_info()
    nc, ns = info.num_cores, info.num_subcores
    nw = nc * ns
    b_per_w = B // nw
    n_chunks = b_per_w // _CHUNK
    assert b_per_w * nw == B and n_chunks * _CHUNK == b_per_w
    assert n_chunks >= 4 and n_chunks % 2 == 0
    mesh = plsc.VectorSubcoreMesh(core_axis_name="c", subcore_axis_name="s")

    @functools.partial(
        pl.kernel,
        mesh=mesh,
        out_type=jax.ShapeDtypeStruct((B, _D), jnp.float32),
        compiler_params=pltpu.CompilerParams(use_tc_tiling_on_sc=False),
        scratch_types=[
            pltpu.VMEM((_CHUNK,), jnp.int32),
            pltpu.VMEM((_CHUNK,), jnp.int32),
            pltpu.VMEM((_CHUNK, _D), jnp.float32),
            pltpu.VMEM((_CHUNK, _D), jnp.float32),
            pltpu.SemaphoreType.DMA,
            pltpu.SemaphoreType.DMA,
            pltpu.SemaphoreType.DMA,
            pltpu.SemaphoreType.DMA,
            pltpu.SemaphoreType.DMA,
            pltpu.SemaphoreType.DMA,
        ],
    )
    def gather_kernel(idx_hbm, table_hbm, out_hbm,
                      idx_v0, idx_v1, rows_v0, rows_v1,
                      isem0, isem1, gsem0, gsem1, wsem0, wsem1):
        wid = lax.axis_index("s") * nc + lax.axis_index("c")
        base = wid * b_per_w
        idx_bufs = (idx_v0, idx_v1)
        rows_bufs = (rows_v0, rows_v1)
        isems = (isem0, isem1)
        gsems = (gsem0, gsem1)
        wsems = (wsem0, wsem1)

        def load_idx(i, s):
            pltpu.async_copy(idx_hbm.at[pl.ds(base + i * _CHUNK, _CHUNK)],
                             idx_bufs[s], isems[s])

        def wait_idx(s):
            pltpu.make_async_copy(idx_hbm.at[pl.ds(base, _CHUNK)],
                                  idx_bufs[s], isems[s]).wait()

        def fire_gather(s):
            pltpu.async_copy(table_hbm.at[idx_bufs[s]], rows_bufs[s],
                             gsems[s])

        def drain_gather(s):
            pltpu.make_async_copy(table_hbm.at[idx_bufs[s]], rows_bufs[s],
                                  gsems[s]).wait()

        def start_wb(i, s):
            pltpu.async_copy(
                rows_bufs[s],
                out_hbm.at[pl.ds(base + i * _CHUNK, _CHUNK)],
                wsems[s],
            )

        def wait_wb(s):
            pltpu.make_async_copy(
                rows_bufs[s], out_hbm.at[pl.ds(base, _CHUNK)],
                wsems[s]).wait()

        def step(i, s, first):
            p = 1 - s
            wait_idx(s)
            if not first:
                wait_wb(s)
            fire_gather(s)
            drain_gather(p)
            load_idx(i + 1, p)
            start_wb(i - 1, p)

        load_idx(0, 0)
        load_idx(1, 1)
        wait_idx(0)
        fire_gather(0)
        step(1, 1, True)
        step(2, 0, False)

        def body(t, carry):
            step(3 + 2 * t, 1, False)
            step(4 + 2 * t, 0, False)
            return carry

        lax.fori_loop(0, (n_chunks - 4) // 2, body, 0)

        wait_idx(1)
        wait_wb(1)
        fire_gather(1)
        drain_gather(0)
        start_wb(n_chunks - 2, 0)
        drain_gather(1)
        start_wb(n_chunks - 1, 1)
        wait_wb(0)
        wait_wb(1)

    return gather_kernel


def kernel(token_ids, weight):
    shape = token_ids.shape
    b = token_ids.size
    idx1d = token_ids.reshape(b).astype(jnp.int32)
    out = _make_gather(b)(idx1d, weight)
    return out.reshape(*shape, _D)
